# Initial kernel scaffold; baseline (speedup 1.0000x reference)
import jax
import jax.numpy as jnp
from jax.experimental import pallas as pl


def kernel(sim_data):
    raise NotImplementedError("write your pallas kernel here")


# speedup vs baseline: 5.4518x; 5.4518x over previous
import dataclasses
import functools

import jax
import jax.numpy as jnp
from jax import lax
from jax.experimental import pallas as pl
from jax.experimental.pallas import tpu as pltpu
from jax.experimental.pallas import tpu_sc as plsc

BINS = 30
NC = 2
NS = 16
L = 16
NW = NC * NS
ROWS = 4096
COLS = 16384


def _sc_compiler_params():
    cp = pltpu.CompilerParams()
    if "needs_layout_passes" in pltpu.CompilerParams.__dataclass_fields__:
        cp = dataclasses.replace(cp, needs_layout_passes=False)
    return cp


def _sc_histogram(x2d):
    mesh = plsc.VectorSubcoreMesh(core_axis_name="c", subcore_axis_name="s")

    @functools.partial(
        pl.kernel,
        out_type=jax.ShapeDtypeStruct((NW, BINS, L), jnp.int32),
        mesh=mesh,
        scratch_types=[pltpu.VMEM((BINS, L), jnp.int32)],
        compiler_params=_sc_compiler_params(),
    )
    def hist_kernel(x_hbm, out_hbm, hist):
        lane = lax.iota(jnp.int32, L)
        ones = jnp.ones((L,), jnp.int32)

        @pl.loop(0, BINS)
        def _(b):
            hist[b, :] = jnp.zeros((L,), jnp.int32)

        def body(blk):
            @pl.loop(0, COLS, step=L)
            def _(i):
                x = blk[0, pl.ds(i, L)]
                t = (x + jnp.float32(1.000001)) * jnp.float32(14.5)
                idx = t.astype(jnp.int32)
                plsc.addupdate_scatter(hist, [idx, lane], ones)

        pltpu.emit_pipeline(
            body,
            grid=(ROWS,),
            in_specs=[pl.BlockSpec((1, COLS), lambda i: (i, 0))],
            core_axis_name=("c", "s"),
            dimension_semantics=(pltpu.PARALLEL,),
        )(x_hbm)

        wid = lax.axis_index("c") * NS + lax.axis_index("s")
        pltpu.sync_copy(hist, out_hbm.at[wid])

    return hist_kernel(x2d)


def _tc_finish(partials):
    def body(c_ref, o_ref):
        counts = jnp.sum(c_ref[...], axis=(0, 2))
        o_ref[...] = jnp.log(counts.astype(jnp.float32) + jnp.float32(1e-5))

    return pl.pallas_call(
        body,
        out_shape=jax.ShapeDtypeStruct((BINS,), jnp.float32),
    )(partials)


def kernel(sim_data):
    x2d = sim_data.reshape(ROWS, COLS)
    partials = _sc_histogram(x2d)
    return _tc_finish(partials)


# speedup vs baseline: 5.8692x; 1.0766x over previous
import dataclasses
import functools

import jax
import jax.numpy as jnp
from jax import lax
from jax.experimental import pallas as pl
from jax.experimental.pallas import tpu as pltpu
from jax.experimental.pallas import tpu_sc as plsc

BINS = 30
NC = 2
NS = 16
L = 16
NW = NC * NS
ROWS = 4096
COLS = 16384


def _sc_compiler_params():
    cp = pltpu.CompilerParams()
    if "needs_layout_passes" in pltpu.CompilerParams.__dataclass_fields__:
        cp = dataclasses.replace(cp, needs_layout_passes=False)
    return cp


def _sc_histogram(x2d):
    mesh = plsc.VectorSubcoreMesh(core_axis_name="c", subcore_axis_name="s")

    @functools.partial(
        pl.kernel,
        out_type=jax.ShapeDtypeStruct((NW, BINS, L), jnp.int32),
        mesh=mesh,
        scratch_types=[pltpu.VMEM((BINS, L), jnp.int32)],
        compiler_params=_sc_compiler_params(),
    )
    def hist_kernel(x_hbm, out_hbm, hist):
        lane = lax.iota(jnp.int32, L)
        ones = jnp.ones((L,), jnp.int32)

        @pl.loop(0, BINS)
        def _(b):
            hist[b, :] = jnp.zeros((L,), jnp.int32)

        def body(blk):
            @pl.loop(0, COLS, step=L, unroll=16)
            def _(i):
                x = blk[0, pl.ds(i, L)]
                t = (x + jnp.float32(1.000001)) * jnp.float32(14.5)
                idx = t.astype(jnp.int32)
                plsc.addupdate_scatter(hist, [idx, lane], ones)

        pltpu.emit_pipeline(
            body,
            grid=(ROWS,),
            in_specs=[pl.BlockSpec((1, COLS), lambda i: (i, 0))],
            core_axis_name=("c", "s"),
            dimension_semantics=(pltpu.PARALLEL,),
        )(x_hbm)

        wid = lax.axis_index("c") * NS + lax.axis_index("s")
        pltpu.sync_copy(hist, out_hbm.at[wid])

    return hist_kernel(x2d)


def _tc_finish(partials):
    def body(c_ref, o_ref):
        counts = jnp.sum(c_ref[...], axis=(0, 2))
        o_ref[...] = jnp.log(counts.astype(jnp.float32) + jnp.float32(1e-5))

    return pl.pallas_call(
        body,
        out_shape=jax.ShapeDtypeStruct((BINS,), jnp.float32),
    )(partials)


def kernel(sim_data):
    x2d = sim_data.reshape(ROWS, COLS)
    partials = _sc_histogram(x2d)
    return _tc_finish(partials)
